# baseline (speedup 1.0000x reference)
import jax
import jax.numpy as jnp
from jax.experimental import pallas as pl
from jax.experimental.pallas import tpu as pltpu

A = 20000
LANES = 128
ROWS = 160
APAD = ROWS * LANES
NUM_OUT = 500
OUT_ROWS = 512
KEPT_ROWS = 4
THRESH = 0.7
NEG_INF = float("-inf")
NB = 2
U = 8


def _nms_body(scores_in, anc_ref, del_ref, out_ref, box_ref, kept_ref):
    for b in range(NB):
        ay1 = anc_ref[b, 0]
        ax1 = anc_ref[b, 1]
        ay2 = anc_ref[b, 2]
        ax2 = anc_ref[b, 3]
        h = ay2 - ay1
        w = ax2 - ax1
        cy = ay1 + 0.5 * h
        cx = ax1 + 0.5 * w
        cy = cy + del_ref[b, 0] * h
        cx = cx + del_ref[b, 1] * w
        h = h * jnp.exp(del_ref[b, 2])
        w = w * jnp.exp(del_ref[b, 3])
        y1 = jnp.clip(cy - 0.5 * h, 0.0, 1.0)
        x1 = jnp.clip(cx - 0.5 * w, 0.0, 1.0)
        y2 = jnp.clip(cy + 0.5 * h, 0.0, 1.0)
        x2 = jnp.clip(cx + 0.5 * w, 0.0, 1.0)
        box_ref[b, 0] = y1
        box_ref[b, 1] = x1
        box_ref[b, 2] = y2
        box_ref[b, 3] = x2
        box_ref[b, 4] = (y2 - y1) * (x2 - x1)

    out_ref[...] = jnp.zeros((NB, OUT_ROWS, 4), jnp.float32)
    kept_ref[...] = jnp.zeros((NB, 5, KEPT_ROWS, LANES), jnp.float32)

    iota2d = (jax.lax.broadcasted_iota(jnp.int32, (ROWS, LANES), 0) * LANES
              + jax.lax.broadcasted_iota(jnp.int32, (ROWS, LANES), 1))
    lane_iota = jax.lax.broadcasted_iota(jnp.int32, (1, LANES), 1)
    row8 = jax.lax.broadcasted_iota(jnp.int32, (U, 1), 0)

    for b in range(NB):
        def cond(carry):
            k, done = carry[0], carry[1]
            return (k < NUM_OUT) & (done == 0)

        def body(carry):
            k, done, scores = carry
            kvec = jnp.reshape(k, (1, 1))

            coords = []
            valids = []
            supk = []
            inv = jnp.zeros((1, 1), jnp.int32)
            for j in range(U):
                m = jnp.max(scores, keepdims=True)
                idx = jnp.min(jnp.where(scores == m, iota2d, APAD),
                              keepdims=True)
                sel = iota2d == idx
                scores = jnp.where(sel, NEG_INF, scores)
                valid = m > NEG_INF
                by1 = jnp.sum(jnp.where(sel, box_ref[b, 0], 0.0),
                              keepdims=True)
                bx1 = jnp.sum(jnp.where(sel, box_ref[b, 1], 0.0),
                              keepdims=True)
                by2 = jnp.sum(jnp.where(sel, box_ref[b, 2], 0.0),
                              keepdims=True)
                bx2 = jnp.sum(jnp.where(sel, box_ref[b, 3], 0.0),
                              keepdims=True)
                area = (by2 - by1) * (bx2 - bx1)
                yy1 = jnp.maximum(by1, kept_ref[b, 0])
                xx1 = jnp.maximum(bx1, kept_ref[b, 1])
                yy2 = jnp.minimum(by2, kept_ref[b, 2])
                xx2 = jnp.minimum(bx2, kept_ref[b, 3])
                inter = (jnp.maximum(yy2 - yy1, 0.0)
                         * jnp.maximum(xx2 - xx1, 0.0))
                union = area + kept_ref[b, 4] - inter
                iou = inter / jnp.maximum(union, 1e-12)
                supk.append(jnp.max(iou, keepdims=True) > THRESH)
                coords.append((by1, bx1, by2, bx2, area))
                valids.append(valid)
                inv = inv | jnp.logical_not(valid).astype(jnp.int32)

            y1r = jnp.concatenate([c[0] for c in coords], axis=1)
            x1r = jnp.concatenate([c[1] for c in coords], axis=1)
            y2r = jnp.concatenate([c[2] for c in coords], axis=1)
            x2r = jnp.concatenate([c[3] for c in coords], axis=1)
            arr = jnp.concatenate([c[4] for c in coords], axis=1)
            y1c = y1r.reshape(U, 1)
            x1c = x1r.reshape(U, 1)
            y2c = y2r.reshape(U, 1)
            x2c = x2r.reshape(U, 1)
            arc = arr.reshape(U, 1)
            pyy1 = jnp.maximum(y1c, y1r)
            pxx1 = jnp.maximum(x1c, x1r)
            pyy2 = jnp.minimum(y2c, y2r)
            pxx2 = jnp.minimum(x2c, x2r)
            pint = (jnp.maximum(pyy2 - pyy1, 0.0)
                    * jnp.maximum(pxx2 - pxx1, 0.0))
            punion = arc + arr - pint
            piou = pint / jnp.maximum(punion, 1e-12)
            sup88 = (piou > THRESH).astype(jnp.int32)

            keepcol = jnp.zeros((U, 1), jnp.int32)
            keeps, prefixes = [], []
            nk = jnp.zeros((1, 1), jnp.int32)
            for j in range(U):
                conflict = jnp.max(
                    jnp.where(row8 < j, keepcol * sup88[:, j:j + 1], 0),
                    keepdims=True)
                keep = (valids[j] & jnp.logical_not(supk[j])
                        & (conflict == 0) & (kvec + nk < NUM_OUT))
                keeps.append(keep)
                prefixes.append(nk)
                keepcol = jnp.where((row8 == j) & keep, 1, keepcol)
                nk = nk + keep.astype(jnp.int32)

            mini = jnp.zeros((U, 4), jnp.float32)
            for j in range(U):
                by1, bx1, by2, bx2, _ = coords[j]
                rowvals = jnp.concatenate([by1, bx1, by2, bx2], axis=1)
                mask = (row8 == prefixes[j]) & keeps[j]
                mini = jnp.where(mask, rowvals, mini)
            out_ref[b, pl.ds(k, U), :] = mini

            krow = k // LANES
            klane = k % LANES
            for off in range(2):
                trow = jnp.minimum(krow + off, KEPT_ROWS - 1)
                masks = [(lane_iota == (klane + prefixes[j] - off * LANES))
                         & keeps[j] for j in range(U)]
                for plane in range(5):
                    rowv = kept_ref[b, plane, pl.ds(trow, 1), :]
                    for j in range(U):
                        rowv = jnp.where(masks[j], coords[j][plane], rowv)
                    kept_ref[b, plane, pl.ds(trow, 1), :] = rowv

            return (k + nk[0, 0], done | inv[0, 0], scores)

        jax.lax.while_loop(cond, body,
                           (jnp.int32(0), jnp.int32(0), scores_in[b]))


@jax.jit
def kernel(rpn_probs, bbox_deltas, anchors):
    B = rpn_probs.shape[0]
    pad = APAD - A
    scores = jnp.pad(rpn_probs[:, :, 1], ((0, 0), (0, pad)),
                     constant_values=NEG_INF).reshape(B, ROWS, LANES)
    anc = jnp.pad(anchors, ((0, 0), (0, pad), (0, 0))).transpose(0, 2, 1)
    anc = anc.reshape(B, 4, ROWS, LANES)
    dlt = jnp.pad(bbox_deltas, ((0, 0), (0, pad), (0, 0))).transpose(0, 2, 1)
    dlt = dlt.reshape(B, 4, ROWS, LANES)

    out = pl.pallas_call(
        _nms_body,
        out_shape=jax.ShapeDtypeStruct((B, OUT_ROWS, 4), jnp.float32),
        scratch_shapes=[
            pltpu.VMEM((NB, 5, ROWS, LANES), jnp.float32),
            pltpu.VMEM((NB, 5, KEPT_ROWS, LANES), jnp.float32),
        ],
    )(scores, anc, dlt)
    return out[:, :NUM_OUT, :]


# speedup vs baseline: 1.7653x; 1.7653x over previous
import jax
import jax.numpy as jnp
from jax.experimental import pallas as pl
from jax.experimental.pallas import tpu as pltpu

A = 20000
LANES = 128
ROWS = 160
APAD = ROWS * LANES
NUM_OUT = 500
OUT_ROWS = 512
THRESH = 0.7
NEG_INF = float("-inf")
NB = 2


def _nms_body(scores_in, anc_ref, del_ref, out_ref, box_ref, sc_ref):
    for b in range(NB):
        ay1 = anc_ref[b, 0]
        ax1 = anc_ref[b, 1]
        ay2 = anc_ref[b, 2]
        ax2 = anc_ref[b, 3]
        h = ay2 - ay1
        w = ax2 - ax1
        cy = ay1 + 0.5 * h
        cx = ax1 + 0.5 * w
        cy = cy + del_ref[b, 0] * h
        cx = cx + del_ref[b, 1] * w
        h = h * jnp.exp(del_ref[b, 2])
        w = w * jnp.exp(del_ref[b, 3])
        y1 = jnp.clip(cy - 0.5 * h, 0.0, 1.0)
        x1 = jnp.clip(cx - 0.5 * w, 0.0, 1.0)
        y2 = jnp.clip(cy + 0.5 * h, 0.0, 1.0)
        x2 = jnp.clip(cx + 0.5 * w, 0.0, 1.0)
        box_ref[b, 0] = y1
        box_ref[b, 1] = x1
        box_ref[b, 2] = y2
        box_ref[b, 3] = x2
        box_ref[b, 4] = (y2 - y1) * (x2 - x1)
        sc_ref[b] = scores_in[b]

    iota2d = (jax.lax.broadcasted_iota(jnp.int32, (ROWS, LANES), 0) * LANES
              + jax.lax.broadcasted_iota(jnp.int32, (ROWS, LANES), 1))
    lane_iota = jax.lax.broadcasted_iota(jnp.int32, (1, LANES), 1)

    summaries = [jnp.max(scores_in[b].reshape(-1, 8, LANES), axis=0)
                 for b in range(NB)]

    def step(i, summaries):
        new_summaries = []
        for b in range(NB):
            scores = sc_ref[b]
            m = jnp.max(summaries[b])
            idx = jnp.min(jnp.where(scores == m, iota2d, APAD))
            valid = m > NEG_INF
            r = idx // LANES
            c = idx % LANES
            lm = lane_iota == c
            by1 = jnp.sum(jnp.where(lm, box_ref[b, 0, pl.ds(r, 1), :], 0.0))
            bx1 = jnp.sum(jnp.where(lm, box_ref[b, 1, pl.ds(r, 1), :], 0.0))
            by2 = jnp.sum(jnp.where(lm, box_ref[b, 2, pl.ds(r, 1), :], 0.0))
            bx2 = jnp.sum(jnp.where(lm, box_ref[b, 3, pl.ds(r, 1), :], 0.0))
            yy1 = jnp.maximum(by1, box_ref[b, 0])
            xx1 = jnp.maximum(bx1, box_ref[b, 1])
            yy2 = jnp.minimum(by2, box_ref[b, 2])
            xx2 = jnp.minimum(bx2, box_ref[b, 3])
            inter = (jnp.maximum(yy2 - yy1, 0.0)
                     * jnp.maximum(xx2 - xx1, 0.0))
            area_b = (by2 - by1) * (bx2 - bx1)
            union = area_b + box_ref[b, 4] - inter
            iou = inter / jnp.maximum(union, 1e-12)
            supp = (iou > THRESH) | (iota2d == idx)
            new_scores = jnp.where(supp, NEG_INF, scores)
            sc_ref[b] = new_scores
            new_summaries.append(
                jnp.max(new_scores.reshape(-1, 8, LANES), axis=0))
            row = jnp.concatenate(
                [by1.reshape(1, 1), bx1.reshape(1, 1),
                 by2.reshape(1, 1), bx2.reshape(1, 1)], axis=1)
            out_ref[b, pl.ds(i, 1), :] = jnp.where(valid, row, 0.0)
        return new_summaries

    jax.lax.fori_loop(0, NUM_OUT, step, summaries)


@jax.jit
def kernel(rpn_probs, bbox_deltas, anchors):
    B = rpn_probs.shape[0]
    pad = APAD - A
    scores = jnp.pad(rpn_probs[:, :, 1], ((0, 0), (0, pad)),
                     constant_values=NEG_INF).reshape(B, ROWS, LANES)
    anc = jnp.pad(anchors, ((0, 0), (0, pad), (0, 0))).transpose(0, 2, 1)
    anc = anc.reshape(B, 4, ROWS, LANES)
    dlt = jnp.pad(bbox_deltas, ((0, 0), (0, pad), (0, 0))).transpose(0, 2, 1)
    dlt = dlt.reshape(B, 4, ROWS, LANES)

    out = pl.pallas_call(
        _nms_body,
        out_shape=jax.ShapeDtypeStruct((B, OUT_ROWS, 4), jnp.float32),
        scratch_shapes=[
            pltpu.VMEM((NB, 5, ROWS, LANES), jnp.float32),
            pltpu.VMEM((NB, ROWS, LANES), jnp.float32),
        ],
    )(scores, anc, dlt)
    return out[:, :NUM_OUT, :]


# speedup vs baseline: 1.8120x; 1.0265x over previous
import jax
import jax.numpy as jnp
from jax.experimental import pallas as pl
from jax.experimental.pallas import tpu as pltpu

A = 20000
LANES = 128
ROWS = 160
APAD = ROWS * LANES
NUM_OUT = 500
OUT_ROWS = 512
THRESH = 0.7
NEG_INF = float("-inf")
NB = 2


def _nms_body(scores_in, anc_ref, del_ref, out_ref, box_ref, sc_ref):
    for b in range(NB):
        ay1 = anc_ref[b, 0]
        ax1 = anc_ref[b, 1]
        ay2 = anc_ref[b, 2]
        ax2 = anc_ref[b, 3]
        h = ay2 - ay1
        w = ax2 - ax1
        cy = ay1 + 0.5 * h
        cx = ax1 + 0.5 * w
        cy = cy + del_ref[b, 0] * h
        cx = cx + del_ref[b, 1] * w
        h = h * jnp.exp(del_ref[b, 2])
        w = w * jnp.exp(del_ref[b, 3])
        y1 = jnp.clip(cy - 0.5 * h, 0.0, 1.0)
        x1 = jnp.clip(cx - 0.5 * w, 0.0, 1.0)
        y2 = jnp.clip(cy + 0.5 * h, 0.0, 1.0)
        x2 = jnp.clip(cx + 0.5 * w, 0.0, 1.0)
        box_ref[b, 0] = y1
        box_ref[b, 1] = x1
        box_ref[b, 2] = y2
        box_ref[b, 3] = x2
        box_ref[b, 4] = (y2 - y1) * (x2 - x1)
        sc_ref[b] = scores_in[b]

    iota2d = (jax.lax.broadcasted_iota(jnp.int32, (ROWS, LANES), 0) * LANES
              + jax.lax.broadcasted_iota(jnp.int32, (ROWS, LANES), 1))
    iota3d = iota2d.reshape(-1, 8, LANES)
    lane_iota = jax.lax.broadcasted_iota(jnp.int32, (1, LANES), 1)

    def _summarize(scores):
        s3 = scores.reshape(-1, 8, LANES)
        sm = jnp.max(s3, axis=0)
        sidx = jnp.min(jnp.where(s3 == sm[None], iota3d, APAD), axis=0)
        return sm, sidx

    summaries = [_summarize(scores_in[b]) for b in range(NB)]

    def step(i, summaries):
        new_summaries = []
        for b in range(NB):
            scores = sc_ref[b]
            sm, sidx = summaries[b]
            m = jnp.max(sm)
            idx = jnp.min(jnp.where(sm == m, sidx, APAD))
            valid = m > NEG_INF
            r = idx // LANES
            c = idx % LANES
            lm = lane_iota == c
            by1 = jnp.sum(jnp.where(lm, box_ref[b, 0, pl.ds(r, 1), :], 0.0))
            bx1 = jnp.sum(jnp.where(lm, box_ref[b, 1, pl.ds(r, 1), :], 0.0))
            by2 = jnp.sum(jnp.where(lm, box_ref[b, 2, pl.ds(r, 1), :], 0.0))
            bx2 = jnp.sum(jnp.where(lm, box_ref[b, 3, pl.ds(r, 1), :], 0.0))
            yy1 = jnp.maximum(by1, box_ref[b, 0])
            xx1 = jnp.maximum(bx1, box_ref[b, 1])
            yy2 = jnp.minimum(by2, box_ref[b, 2])
            xx2 = jnp.minimum(bx2, box_ref[b, 3])
            inter = (jnp.maximum(yy2 - yy1, 0.0)
                     * jnp.maximum(xx2 - xx1, 0.0))
            area_b = (by2 - by1) * (bx2 - bx1)
            union = area_b + box_ref[b, 4] - inter
            iou = inter / jnp.maximum(union, 1e-12)
            supp = (iou > THRESH) | (iota2d == idx)
            new_scores = jnp.where(supp, NEG_INF, scores)
            sc_ref[b] = new_scores
            new_summaries.append(_summarize(new_scores))
            row = jnp.concatenate(
                [by1.reshape(1, 1), bx1.reshape(1, 1),
                 by2.reshape(1, 1), bx2.reshape(1, 1)], axis=1)
            out_ref[b, pl.ds(i, 1), :] = jnp.where(valid, row, 0.0)
        return new_summaries

    jax.lax.fori_loop(0, NUM_OUT, step, summaries)


@jax.jit
def kernel(rpn_probs, bbox_deltas, anchors):
    B = rpn_probs.shape[0]
    pad = APAD - A
    scores = jnp.pad(rpn_probs[:, :, 1], ((0, 0), (0, pad)),
                     constant_values=NEG_INF).reshape(B, ROWS, LANES)
    anc = jnp.pad(anchors, ((0, 0), (0, pad), (0, 0))).transpose(0, 2, 1)
    anc = anc.reshape(B, 4, ROWS, LANES)
    dlt = jnp.pad(bbox_deltas, ((0, 0), (0, pad), (0, 0))).transpose(0, 2, 1)
    dlt = dlt.reshape(B, 4, ROWS, LANES)

    out = pl.pallas_call(
        _nms_body,
        out_shape=jax.ShapeDtypeStruct((B, OUT_ROWS, 4), jnp.float32),
        scratch_shapes=[
            pltpu.VMEM((NB, 5, ROWS, LANES), jnp.float32),
            pltpu.VMEM((NB, ROWS, LANES), jnp.float32),
        ],
    )(scores, anc, dlt)
    return out[:, :NUM_OUT, :]
